# baseline (speedup 1.0000x reference)
import jax
import jax.numpy as jnp
from jax import lax
from jax.experimental import pallas as pl
from jax.experimental.pallas import tpu as pltpu
from jax.experimental.pallas import tpu_sc as plsc

NUM_USERS = 10000
NUM_ITEMS = 40000
N = NUM_USERS + NUM_ITEMS
EMB = 64
E = 800000

W = 16
NPAD = 50176
RPT = NPAD // 16
EPAD = 819200
ER = EPAD // 128
ROWS_TILE = ER // 16
CH = 16
NCH = ROWS_TILE // CH
BE = 512
GB = CH * 128 // BE
DEG_CH = 8
DEG_ROWS_TILE = (ER // 2) // 16
DEG_NCH = DEG_ROWS_TILE // DEG_CH
NBUF = 3


def _sc_scatter(x0, x1, x2, x3, x4, x5, x6, x7, srcf, dstf, zrows, ones,
                a0, a1, a2, a3, a4, a5, a6, a7, dga, dgb,
                src_v, dst_v, r0_v, r1_v, r2_v, table, acc,
                gsem, ssem):
    c = lax.axis_index("c")
    s = lax.axis_index("s")
    rows = [r0_v, r1_v, r2_v]

    def zero_acc():
        pltpu.sync_copy(zrows.at[pl.ds(s * RPT, RPT)],
                        acc.at[pl.ds(s * RPT, RPT)])

    def readback(outref):
        pltpu.sync_copy(acc.at[pl.ds(s * RPT, RPT)],
                        outref.at[pl.ds(s * RPT, RPT)])

    def chunk_pass(xref, outref):
        pltpu.sync_copy(xref.at[pl.ds(s * RPT, RPT)],
                        table.at[pl.ds(s * RPT, RPT)])
        zero_acc()
        plsc.subcore_barrier()
        base = s * ROWS_TILE

        def body(i, carry):
            e0 = (base + i * CH) * 128
            pltpu.sync_copy(srcf.at[pl.ds(e0, CH * 128)], src_v)
            pltpu.sync_copy(dstf.at[pl.ds(e0, CH * 128)], dst_v)
            g = {}
            sc = {}

            def fire_scatter(b):
                g[b].wait()
                sc[b] = pltpu.async_copy(
                    rows[b % NBUF],
                    acc.at[dst_v.at[pl.ds(b * BE, BE)]], ssem, add=True)

            for b in range(GB):
                if b >= NBUF:
                    sc[b - NBUF].wait()
                g[b] = pltpu.async_copy(
                    table.at[src_v.at[pl.ds(b * BE, BE)]],
                    rows[b % NBUF], gsem)
                if b >= 1:
                    fire_scatter(b - 1)
            fire_scatter(GB - 1)
            for b in range(max(GB - NBUF, 0), GB):
                sc[b].wait()
            return carry

        lax.fori_loop(0, NCH, body, 0)
        plsc.subcore_barrier()
        readback(outref)

    def deg_pass(outref, lo):
        zero_acc()
        pltpu.sync_copy(ones, r0_v)
        plsc.subcore_barrier()
        base = lo + s * DEG_ROWS_TILE

        def body(i, carry):
            e0 = (base + i * DEG_CH) * 128
            pltpu.sync_copy(dstf.at[pl.ds(e0, DEG_CH * 128)],
                            dst_v.at[pl.ds(0, DEG_CH * 128)])
            sc = [pltpu.async_copy(
                      r0_v,
                      acc.at[dst_v.at[pl.ds(b * BE, BE)]], ssem, add=True)
                  for b in range(DEG_CH * 128 // BE)]
            for cp in sc:
                cp.wait()
            return carry

        lax.fori_loop(0, DEG_NCH, body, 0)
        plsc.subcore_barrier()
        readback(outref)

    @pl.when(c == 0)
    def _():
        chunk_pass(x0, a0)
        chunk_pass(x1, a1)
        chunk_pass(x2, a2)
        chunk_pass(x3, a3)
        deg_pass(dga, 0)

    @pl.when(c == 1)
    def _():
        chunk_pass(x4, a4)
        chunk_pass(x5, a5)
        chunk_pass(x6, a6)
        chunk_pass(x7, a7)
        deg_pass(dgb, ER // 2)


def _sc_call(xc, srcf, dstr, zrows, ones):
    f32 = jnp.float32
    out_type = tuple(jax.ShapeDtypeStruct((NPAD, W), f32) for _ in range(10))
    mesh = plsc.VectorSubcoreMesh(core_axis_name="c", subcore_axis_name="s")
    kern = pl.kernel(
        _sc_scatter, out_type=out_type, mesh=mesh,
        compiler_params=pltpu.CompilerParams(use_tc_tiling_on_sc=False),
        scratch_types=[
            pltpu.VMEM((CH * 128,), jnp.int32),
            pltpu.VMEM((CH * 128,), jnp.int32),
            pltpu.VMEM((BE, W), f32),
            pltpu.VMEM((BE, W), f32),
            pltpu.VMEM((BE, W), f32),
            pltpu.VMEM_SHARED((NPAD, W), f32),
            pltpu.VMEM_SHARED((NPAD, W), f32),
            pltpu.SemaphoreType.DMA,
            pltpu.SemaphoreType.DMA,
        ],
    )
    return kern(*xc, srcf, dstr, zrows, ones)


_BLK_I = 2000
_BLK_N = 2000


def _lrelu(x):
    return jnp.where(x >= 0, x, 0.01 * x)


def _proj_body(f0, f1, w0, b0, w1, b1, p0, p1):
    a0 = jnp.dot(f0[...], w0[...], preferred_element_type=jnp.float32) + b0[...]
    a1 = jnp.dot(f1[...], w1[...], preferred_element_type=jnp.float32) + b1[...]
    p0[...] = _lrelu(a0)
    p1[...] = _lrelu(a1)


def _project(feat0, feat1, W0, b0, W1, b1):
    D0 = feat0.shape[1]
    D1 = feat1.shape[1]
    grid = NUM_ITEMS // _BLK_I
    return pl.pallas_call(
        _proj_body,
        grid=(grid,),
        in_specs=[
            pl.BlockSpec((_BLK_I, D0), lambda i: (i, 0)),
            pl.BlockSpec((_BLK_I, D1), lambda i: (i, 0)),
            pl.BlockSpec((D0, EMB), lambda i: (0, 0)),
            pl.BlockSpec((1, EMB), lambda i: (0, 0)),
            pl.BlockSpec((D1, EMB), lambda i: (0, 0)),
            pl.BlockSpec((1, EMB), lambda i: (0, 0)),
        ],
        out_specs=[
            pl.BlockSpec((_BLK_I, EMB), lambda i: (i, 0)),
            pl.BlockSpec((_BLK_I, EMB), lambda i: (i, 0)),
        ],
        out_shape=[
            jax.ShapeDtypeStruct((NUM_ITEMS, EMB), jnp.float32),
            jax.ShapeDtypeStruct((NUM_ITEMS, EMB), jnp.float32),
        ],
    )(feat0, feat1, W0, b0.reshape(1, EMB), W1, b1.reshape(1, EMB))


def _comb_body(a0, a1, a2, a3, a4, a5, a6, a7, dga, dgb, ne, wg0, wg1, wid,
               out):
    deg = jnp.maximum(dga[:, 0:1] + dgb[:, 0:1], 1.0)
    agg0 = jnp.concatenate([a0[...], a1[...], a2[...], a3[...]], axis=1) / deg
    agg1 = jnp.concatenate([a4[...], a5[...], a6[...], a7[...]], axis=1) / deg
    nev = ne[...]
    idp = jnp.dot(nev, wid[...], preferred_element_type=jnp.float32)
    h0 = _lrelu(jnp.dot(agg0, wg0[...], preferred_element_type=jnp.float32) + idp)
    h1 = _lrelu(jnp.dot(agg1, wg1[...], preferred_element_type=jnp.float32) + idp)
    out[...] = h0 + h1 + nev


def _combine(aggs, dga, dgb, node_emb, Wg0, Wg1, Wid):
    grid = N // _BLK_N
    cspec = pl.BlockSpec((_BLK_N, W), lambda i: (i, 0))
    wspec = pl.BlockSpec((EMB, EMB), lambda i: (0, 0))
    return pl.pallas_call(
        _comb_body,
        grid=(grid,),
        in_specs=[cspec] * 10 + [
            pl.BlockSpec((_BLK_N, EMB), lambda i: (i, 0)),
            wspec, wspec, wspec],
        out_specs=pl.BlockSpec((_BLK_N, EMB), lambda i: (i, 0)),
        out_shape=jax.ShapeDtypeStruct((N, EMB), jnp.float32),
    )(*aggs, dga, dgb, node_emb, Wg0, Wg1, Wid)


def kernel(node_emb, feat0, feat1, user_pref0, user_pref1, edge_index,
           W_proj0, b_proj0, W_proj1, b_proj1, W_gcn0, W_gcn1, W_id):
    p0, p1 = _project(feat0, feat1, W_proj0, b_proj0, W_proj1, b_proj1)
    x0 = jnp.concatenate([user_pref0, p0], axis=0)
    x1 = jnp.concatenate([user_pref1, p1], axis=0)
    pres = jnp.stack([x0, x1])

    npad_e = EPAD - E
    src = jnp.concatenate([edge_index[0], jnp.zeros((npad_e,), jnp.int32)])
    dst = jnp.concatenate([edge_index[1],
                           jnp.full((npad_e,), N, jnp.int32)])
    zrows = jnp.zeros((NPAD, W), jnp.float32)
    ones = jnp.ones((BE, W), jnp.float32)
    rpad = jnp.zeros((NPAD - N, W), jnp.float32)
    xc = [jnp.concatenate([x[:, k * W:(k + 1) * W], rpad], axis=0)
          for x in (x0, x1) for k in range(4)]

    outs = _sc_call(xc, src, dst, zrows, ones)
    aggs, dga, dgb = outs[:8], outs[8], outs[9]

    emb = _combine(aggs, dga, dgb, node_emb, W_gcn0, W_gcn1, W_id)
    return emb[:NUM_USERS], emb[NUM_USERS:], node_emb, pres


# speedup vs baseline: 1.5054x; 1.5054x over previous
import jax
import jax.numpy as jnp
from jax import lax
from jax.experimental import pallas as pl
from jax.experimental.pallas import tpu as pltpu
from jax.experimental.pallas import tpu_sc as plsc

NUM_USERS = 10000
NUM_ITEMS = 40000
N = NUM_USERS + NUM_ITEMS
EMB = 64
E = 800000

W = 16
NACC = 50048
TS = 3128
TS_LAST = N - 15 * TS
OUTW = 160
EPAD = 819200
ER = EPAD // 128
ROWS_TILE = ER // 16
CH = 16
NCH = ROWS_TILE // CH
BE = 512
GB = CH * 128 // BE
DEG_CH = 8
DEG_ROWS_TILE = (ER // 2) // 16
DEG_NCH = DEG_ROWS_TILE // DEG_CH
NBUF = 3


def _sc_scatter(xin, srcf, dstf, zrows, ones, out,
                src_v, dst_v, r0_v, r1_v, r2_v, table, acc,
                gsem, ssem):
    c = lax.axis_index("c")
    s = lax.axis_index("s")
    rows = [r0_v, r1_v, r2_v]

    def per_tile_rows(body_fn):
        @pl.when(s < 15)
        def _():
            body_fn(s * TS, TS)

        @pl.when(s == 15)
        def _():
            body_fn(15 * TS, TS_LAST)

    def zero_acc():
        def body(r0, n):
            pltpu.sync_copy(zrows.at[pl.ds(0, n)], acc.at[pl.ds(r0, n)])
        per_tile_rows(body)

    def readback(col):
        def body(r0, n):
            pltpu.sync_copy(acc.at[pl.ds(r0, n)],
                            out.at[pl.ds(r0, n), pl.ds(col, W)])
        per_tile_rows(body)

    def edge_loop(gather):
        base = lax.axis_index("s") * ROWS_TILE

        def body(i, carry):
            e0 = (base + i * CH) * 128
            if gather:
                pltpu.sync_copy(srcf.at[pl.ds(e0, CH * 128)], src_v)
            pltpu.sync_copy(dstf.at[pl.ds(e0, CH * 128)], dst_v)
            g = {}
            sc = {}

            def fire_scatter(b):
                if gather:
                    g[b].wait()
                src_buf = rows[b % NBUF] if gather else r0_v
                sc[b] = pltpu.async_copy(
                    src_buf,
                    acc.at[dst_v.at[pl.ds(b * BE, BE)]], ssem, add=True)

            for b in range(GB):
                if gather:
                    if b >= NBUF:
                        sc[b - NBUF].wait()
                    g[b] = pltpu.async_copy(
                        table.at[src_v.at[pl.ds(b * BE, BE)]],
                        rows[b % NBUF], gsem)
                    if b >= 1:
                        fire_scatter(b - 1)
                else:
                    fire_scatter(b)
            if gather:
                fire_scatter(GB - 1)
                for b in range(max(GB - NBUF, 0), GB):
                    sc[b].wait()
            else:
                for b in range(GB):
                    sc[b].wait()
            return carry

        return body

    def chunk_pass(k):
        def stage(r0, n):
            pltpu.sync_copy(xin.at[pl.ds(r0, n), pl.ds(k * W, W)],
                            table.at[pl.ds(r0, n)])
        per_tile_rows(stage)
        zero_acc()
        plsc.subcore_barrier()
        lax.fori_loop(0, NCH, edge_loop(True), 0)
        plsc.subcore_barrier()
        readback(k * W)

    def deg_pass(col, lo):
        zero_acc()
        pltpu.sync_copy(ones, r0_v)
        plsc.subcore_barrier()
        base = lo + s * DEG_ROWS_TILE

        def body(i, carry):
            e0 = (base + i * DEG_CH) * 128
            pltpu.sync_copy(dstf.at[pl.ds(e0, DEG_CH * 128)],
                            dst_v.at[pl.ds(0, DEG_CH * 128)])
            sc = [pltpu.async_copy(
                      r0_v,
                      acc.at[dst_v.at[pl.ds(b * BE, BE)]], ssem, add=True)
                  for b in range(DEG_CH * 128 // BE)]
            for cp in sc:
                cp.wait()
            return carry

        lax.fori_loop(0, DEG_NCH, body, 0)
        plsc.subcore_barrier()
        readback(col)

    @pl.when(c == 0)
    def _():
        for k in range(4):
            chunk_pass(k)
        deg_pass(128, 0)

    @pl.when(c == 1)
    def _():
        for k in range(4, 8):
            chunk_pass(k)
        deg_pass(144, ER // 2)


def _sc_call(xin, srcf, dstf, zrows, ones):
    f32 = jnp.float32
    mesh = plsc.VectorSubcoreMesh(core_axis_name="c", subcore_axis_name="s")
    kern = pl.kernel(
        _sc_scatter,
        out_type=jax.ShapeDtypeStruct((N, OUTW), f32),
        mesh=mesh,
        compiler_params=pltpu.CompilerParams(use_tc_tiling_on_sc=False),
        scratch_types=[
            pltpu.VMEM((CH * 128,), jnp.int32),
            pltpu.VMEM((CH * 128,), jnp.int32),
            pltpu.VMEM((BE, W), f32),
            pltpu.VMEM((BE, W), f32),
            pltpu.VMEM((BE, W), f32),
            pltpu.VMEM_SHARED((NACC, W), f32),
            pltpu.VMEM_SHARED((NACC, W), f32),
            pltpu.SemaphoreType.DMA,
            pltpu.SemaphoreType.DMA,
        ],
    )
    return kern(xin, srcf, dstf, zrows, ones)


_BLK_I = 2000
_BLK_N = 2000


def _lrelu(x):
    return jnp.where(x >= 0, x, 0.01 * x)


def _proj_body(f0, f1, w0, b0, w1, b1, p0, p1):
    a0 = jnp.dot(f0[...], w0[...], preferred_element_type=jnp.float32) + b0[...]
    a1 = jnp.dot(f1[...], w1[...], preferred_element_type=jnp.float32) + b1[...]
    p0[...] = _lrelu(a0)
    p1[...] = _lrelu(a1)


def _project(feat0, feat1, W0, b0, W1, b1):
    D0 = feat0.shape[1]
    D1 = feat1.shape[1]
    grid = NUM_ITEMS // _BLK_I
    return pl.pallas_call(
        _proj_body,
        grid=(grid,),
        in_specs=[
            pl.BlockSpec((_BLK_I, D0), lambda i: (i, 0)),
            pl.BlockSpec((_BLK_I, D1), lambda i: (i, 0)),
            pl.BlockSpec((D0, EMB), lambda i: (0, 0)),
            pl.BlockSpec((1, EMB), lambda i: (0, 0)),
            pl.BlockSpec((D1, EMB), lambda i: (0, 0)),
            pl.BlockSpec((1, EMB), lambda i: (0, 0)),
        ],
        out_specs=[
            pl.BlockSpec((_BLK_I, EMB), lambda i: (i, 0)),
            pl.BlockSpec((_BLK_I, EMB), lambda i: (i, 0)),
        ],
        out_shape=[
            jax.ShapeDtypeStruct((NUM_ITEMS, EMB), jnp.float32),
            jax.ShapeDtypeStruct((NUM_ITEMS, EMB), jnp.float32),
        ],
    )(feat0, feat1, W0, b0.reshape(1, EMB), W1, b1.reshape(1, EMB))


def _comb_body(ad, ne, wg0, wg1, wid, out):
    a = ad[...]
    deg = jnp.maximum(a[:, 128:129] + a[:, 144:145], 1.0)
    agg0 = a[:, 0:64] / deg
    agg1 = a[:, 64:128] / deg
    nev = ne[...]
    idp = jnp.dot(nev, wid[...], preferred_element_type=jnp.float32)
    h0 = _lrelu(jnp.dot(agg0, wg0[...], preferred_element_type=jnp.float32) + idp)
    h1 = _lrelu(jnp.dot(agg1, wg1[...], preferred_element_type=jnp.float32) + idp)
    out[...] = h0 + h1 + nev


def _combine(aggdeg, node_emb, Wg0, Wg1, Wid):
    grid = N // _BLK_N
    wspec = pl.BlockSpec((EMB, EMB), lambda i: (0, 0))
    return pl.pallas_call(
        _comb_body,
        grid=(grid,),
        in_specs=[pl.BlockSpec((_BLK_N, OUTW), lambda i: (i, 0)),
                  pl.BlockSpec((_BLK_N, EMB), lambda i: (i, 0)),
                  wspec, wspec, wspec],
        out_specs=pl.BlockSpec((_BLK_N, EMB), lambda i: (i, 0)),
        out_shape=jax.ShapeDtypeStruct((N, EMB), jnp.float32),
    )(aggdeg, node_emb, Wg0, Wg1, Wid)


def kernel(node_emb, feat0, feat1, user_pref0, user_pref1, edge_index,
           W_proj0, b_proj0, W_proj1, b_proj1, W_gcn0, W_gcn1, W_id):
    p0, p1 = _project(feat0, feat1, W_proj0, b_proj0, W_proj1, b_proj1)
    x0 = jnp.concatenate([user_pref0, p0], axis=0)
    x1 = jnp.concatenate([user_pref1, p1], axis=0)
    pres = jnp.stack([x0, x1])
    xin = jnp.concatenate([x0, x1], axis=1)

    npad_e = EPAD - E
    src = jnp.concatenate([edge_index[0], jnp.zeros((npad_e,), jnp.int32)])
    dst = jnp.concatenate([edge_index[1],
                           jnp.full((npad_e,), N, jnp.int32)])

    zrows = jnp.zeros((TS, W), jnp.float32)
    ones = jnp.ones((BE, W), jnp.float32)

    aggdeg = _sc_call(xin, src, dst, zrows, ones)
    emb = _combine(aggdeg, node_emb, W_gcn0, W_gcn1, W_id)
    return emb[:NUM_USERS], emb[NUM_USERS:], node_emb, pres


# speedup vs baseline: 1.5465x; 1.0273x over previous
import jax
import jax.numpy as jnp
from jax import lax
from jax.experimental import pallas as pl
from jax.experimental.pallas import tpu as pltpu
from jax.experimental.pallas import tpu_sc as plsc

NUM_USERS = 10000
NUM_ITEMS = 40000
N = NUM_USERS + NUM_ITEMS
EMB = 64
E = 800000

W = 16
NACC = 50048
TS = 3128
TS_LAST = N - 15 * TS
OUTW = 160
EPAD = 819200
ER = EPAD // 128
ROWS_TILE = ER // 16
CH = 16
NCH = ROWS_TILE // CH
BE = 512
GB = CH * 128 // BE
DEG_CH = 8
DEG_ROWS_TILE = (ER // 2) // 16
DEG_NCH = DEG_ROWS_TILE // DEG_CH
NBUF = 3


def _sc_scatter(xin, srcf, dstf, zrows, ones, out,
                src_v, dst_v, r0_v, r1_v, r2_v, table, acc,
                gsem, ssem):
    c = lax.axis_index("c")
    s = lax.axis_index("s")
    rows = [r0_v, r1_v, r2_v]

    def per_tile_rows(body_fn):
        @pl.when(s < 15)
        def _():
            body_fn(s * TS, TS)

        @pl.when(s == 15)
        def _():
            body_fn(15 * TS, TS_LAST)

    def zero_acc():
        def body(r0, n):
            pltpu.sync_copy(zrows.at[pl.ds(0, n)], acc.at[pl.ds(r0, n)])
        per_tile_rows(body)

    def readback(col):
        def body(r0, n):
            pltpu.sync_copy(acc.at[pl.ds(r0, n)],
                            out.at[pl.ds(r0, n), pl.ds(col, W)])
        per_tile_rows(body)

    def edge_loop(gather):
        base = lax.axis_index("s") * ROWS_TILE

        def body(i, carry):
            e0 = (base + i * CH) * 128
            if gather:
                pltpu.sync_copy(srcf.at[pl.ds(e0, CH * 128)], src_v)
            pltpu.sync_copy(dstf.at[pl.ds(e0, CH * 128)], dst_v)
            g = {}
            sc = {}

            def fire_scatter(b):
                if gather:
                    g[b].wait()
                src_buf = rows[b % NBUF] if gather else r0_v
                sc[b] = pltpu.async_copy(
                    src_buf,
                    acc.at[dst_v.at[pl.ds(b * BE, BE)]], ssem, add=True)

            for b in range(GB):
                if gather:
                    if b >= NBUF:
                        sc[b - NBUF].wait()
                    g[b] = pltpu.async_copy(
                        table.at[src_v.at[pl.ds(b * BE, BE)]],
                        rows[b % NBUF], gsem)
                    if b >= 1:
                        fire_scatter(b - 1)
                else:
                    fire_scatter(b)
            if gather:
                fire_scatter(GB - 1)
                for b in range(max(GB - NBUF, 0), GB):
                    sc[b].wait()
            else:
                for b in range(GB):
                    sc[b].wait()
            return carry

        return body

    def chunk_pass(k):
        def stage(r0, n):
            pltpu.sync_copy(xin.at[pl.ds(r0, n), pl.ds(k * W, W)],
                            table.at[pl.ds(r0, n)])
        per_tile_rows(stage)
        zero_acc()
        plsc.subcore_barrier()
        lax.fori_loop(0, NCH, edge_loop(True), 0)
        plsc.subcore_barrier()
        readback(k * W)

    def deg_pass(col, lo):
        zero_acc()
        pltpu.sync_copy(ones, r0_v)
        plsc.subcore_barrier()
        base = lo + s * DEG_ROWS_TILE

        def body(i, carry):
            e0 = (base + i * DEG_CH) * 128
            pltpu.sync_copy(dstf.at[pl.ds(e0, DEG_CH * 128)],
                            dst_v.at[pl.ds(0, DEG_CH * 128)])
            sc = [pltpu.async_copy(
                      r0_v,
                      acc.at[dst_v.at[pl.ds(b * BE, BE)]], ssem, add=True)
                  for b in range(DEG_CH * 128 // BE)]
            for cp in sc:
                cp.wait()
            return carry

        lax.fori_loop(0, DEG_NCH, body, 0)
        plsc.subcore_barrier()
        readback(col)

    @pl.when(c == 0)
    def _():
        for k in range(4):
            chunk_pass(k)
        deg_pass(128, 0)

    @pl.when(c == 1)
    def _():
        for k in range(4, 8):
            chunk_pass(k)
        deg_pass(144, ER // 2)


def _sc_call(xin, srcf, dstf, zrows, ones):
    f32 = jnp.float32
    mesh = plsc.VectorSubcoreMesh(core_axis_name="c", subcore_axis_name="s")
    kern = pl.kernel(
        _sc_scatter,
        out_type=jax.ShapeDtypeStruct((N, OUTW), f32),
        mesh=mesh,
        compiler_params=pltpu.CompilerParams(use_tc_tiling_on_sc=False),
        scratch_types=[
            pltpu.VMEM((CH * 128,), jnp.int32),
            pltpu.VMEM((CH * 128,), jnp.int32),
            pltpu.VMEM((BE, W), f32),
            pltpu.VMEM((BE, W), f32),
            pltpu.VMEM((BE, W), f32),
            pltpu.VMEM_SHARED((NACC, W), f32),
            pltpu.VMEM_SHARED((NACC, W), f32),
            pltpu.SemaphoreType.DMA,
            pltpu.SemaphoreType.DMA,
        ],
    )
    return kern(xin, srcf, dstf, zrows, ones)


_BLK_I = 2000
_BLK_N = 2000


def _lrelu(x):
    return jnp.where(x >= 0, x, 0.01 * x)


_UB = NUM_USERS // _BLK_I


def _build_body(f0, f1, u0, u1, w0, b0, w1, b1, xin, pres):
    i = pl.program_id(0)

    @pl.when(i < _UB)
    def _():
        v0 = u0[...]
        v1 = u1[...]
        xin[...] = jnp.concatenate([v0, v1], axis=1)
        pres[0] = v0
        pres[1] = v1

    @pl.when(i >= _UB)
    def _():
        a0 = _lrelu(jnp.dot(f0[...], w0[...],
                            preferred_element_type=jnp.float32) + b0[...])
        a1 = _lrelu(jnp.dot(f1[...], w1[...],
                            preferred_element_type=jnp.float32) + b1[...])
        xin[...] = jnp.concatenate([a0, a1], axis=1)
        pres[0] = a0
        pres[1] = a1


def _build(feat0, feat1, upref0, upref1, W0, b0, W1, b1):
    D0 = feat0.shape[1]
    D1 = feat1.shape[1]
    grid = N // _BLK_I
    fspec0 = pl.BlockSpec((_BLK_I, D0),
                          lambda i: (jnp.maximum(i - _UB, 0), 0))
    fspec1 = pl.BlockSpec((_BLK_I, D1),
                          lambda i: (jnp.maximum(i - _UB, 0), 0))
    uspec = pl.BlockSpec((_BLK_I, EMB),
                         lambda i: (jnp.minimum(i, _UB - 1), 0))
    return pl.pallas_call(
        _build_body,
        grid=(grid,),
        in_specs=[
            fspec0, fspec1, uspec, uspec,
            pl.BlockSpec((D0, EMB), lambda i: (0, 0)),
            pl.BlockSpec((1, EMB), lambda i: (0, 0)),
            pl.BlockSpec((D1, EMB), lambda i: (0, 0)),
            pl.BlockSpec((1, EMB), lambda i: (0, 0)),
        ],
        out_specs=[
            pl.BlockSpec((_BLK_I, 2 * EMB), lambda i: (i, 0)),
            pl.BlockSpec((2, _BLK_I, EMB), lambda i: (0, i, 0)),
        ],
        out_shape=[
            jax.ShapeDtypeStruct((N, 2 * EMB), jnp.float32),
            jax.ShapeDtypeStruct((2, N, EMB), jnp.float32),
        ],
    )(feat0, feat1, upref0, upref1,
      W0, b0.reshape(1, EMB), W1, b1.reshape(1, EMB))


def _comb_body(ad, ne, wg0, wg1, wid, out):
    a = ad[...]
    deg = jnp.maximum(a[:, 128:129] + a[:, 144:145], 1.0)
    agg0 = a[:, 0:64] / deg
    agg1 = a[:, 64:128] / deg
    nev = ne[...]
    idp = jnp.dot(nev, wid[...], preferred_element_type=jnp.float32)
    h0 = _lrelu(jnp.dot(agg0, wg0[...], preferred_element_type=jnp.float32) + idp)
    h1 = _lrelu(jnp.dot(agg1, wg1[...], preferred_element_type=jnp.float32) + idp)
    out[...] = h0 + h1 + nev


def _combine(aggdeg, node_emb, Wg0, Wg1, Wid):
    grid = N // _BLK_N
    wspec = pl.BlockSpec((EMB, EMB), lambda i: (0, 0))
    return pl.pallas_call(
        _comb_body,
        grid=(grid,),
        in_specs=[pl.BlockSpec((_BLK_N, OUTW), lambda i: (i, 0)),
                  pl.BlockSpec((_BLK_N, EMB), lambda i: (i, 0)),
                  wspec, wspec, wspec],
        out_specs=pl.BlockSpec((_BLK_N, EMB), lambda i: (i, 0)),
        out_shape=jax.ShapeDtypeStruct((N, EMB), jnp.float32),
    )(aggdeg, node_emb, Wg0, Wg1, Wid)


def kernel(node_emb, feat0, feat1, user_pref0, user_pref1, edge_index,
           W_proj0, b_proj0, W_proj1, b_proj1, W_gcn0, W_gcn1, W_id):
    xin, pres = _build(feat0, feat1, user_pref0, user_pref1,
                       W_proj0, b_proj0, W_proj1, b_proj1)

    npad_e = EPAD - E
    src = jnp.concatenate([edge_index[0], jnp.zeros((npad_e,), jnp.int32)])
    dst = jnp.concatenate([edge_index[1],
                           jnp.full((npad_e,), N, jnp.int32)])

    zrows = jnp.zeros((TS, W), jnp.float32)
    ones = jnp.ones((BE, W), jnp.float32)

    aggdeg = _sc_call(xin, src, dst, zrows, ones)
    emb = _combine(aggdeg, node_emb, W_gcn0, W_gcn1, W_id)
    return emb[:NUM_USERS], emb[NUM_USERS:], node_emb, pres
